# baseline (speedup 1.0000x reference)
import jax
import jax.numpy as jnp
import numpy as np
from jax.experimental import pallas as pl

_T = 128
_D = 64
_TOPK = 8
_MAXR = 255.0

_BB = 64


def _make_d8m():
    i = np.arange(_T)
    d = np.abs(i[:, None] - i[None, :]).astype(np.float32)
    decay = np.float32(1.0) - (np.float32(0.1) * d) / np.float32(128.0)
    tril = i[None, :] <= i[:, None]
    return np.where(tril, decay * np.float32(0.125),
                    np.float32(0.0)).astype(np.float32)


_D8M = _make_d8m()


def _half(x, wq, wk, wv, d8m, inv_g):
    hb = x.shape[0]
    x2 = x.reshape(hb * _T, _D)
    q = jnp.dot(x2, wq, preferred_element_type=jnp.float32)
    k = jnp.dot(x2, wk, preferred_element_type=jnp.float32)
    v = jnp.dot(x2, wv, preferred_element_type=jnp.float32)
    q = q.reshape(hb, _T, _D)
    k = k.reshape(hb, _T, _D)
    v = v.reshape(hb, _T, _D) * inv_g

    s = jax.lax.dot_general(
        q, k, (((2,), (2,)), ((0,), (0,))),
        preferred_element_type=jnp.float32)

    f = jnp.maximum(s, 0.0) * d8m


    thresh = jnp.max(f, axis=-1, keepdims=True)
    m = thresh
    for _ in range(_TOPK - 1):
        thresh = jnp.max(jnp.where(f < thresh, f, 0.0), axis=-1, keepdims=True)

    denom = m + 1e-6
    r = _MAXR / denom
    norm = jnp.floor(f * r)
    w = jnp.where(f >= thresh, norm - jnp.where(norm > 127.5, 256.0, 0.0),
                  0.0)

    return jax.lax.dot_general(
        w, v, (((2,), (1,)), ((0,), (0,))),
        preferred_element_type=jnp.float32)


def _head_body(x_ref, wq_ref, wk_ref, wv_ref, d8m_ref, g_ref, out_ref):
    wq = wq_ref[...]
    wk = wk_ref[...]
    wv = wv_ref[...]
    d8m = d8m_ref[...][None]
    inv_g = 1.0 / g_ref[0, 0]
    hb = _BB // 2
    out_ref[:hb] = _half(x_ref[:hb], wq, wk, wv, d8m, inv_g)
    out_ref[hb:] = _half(x_ref[hb:], wq, wk, wv, d8m, inv_g)


def kernel(x, Wk, Wq, Wv, gamma):
    b, t, d = x.shape
    g = jnp.reshape(gamma, (1, 1)).astype(jnp.float32)
    return pl.pallas_call(
        _head_body,
        grid=(b // _BB,),
        in_specs=[
            pl.BlockSpec((_BB, t, d), lambda i: (i, 0, 0)),
            pl.BlockSpec((d, d), lambda i: (0, 0)),
            pl.BlockSpec((d, d), lambda i: (0, 0)),
            pl.BlockSpec((d, d), lambda i: (0, 0)),
            pl.BlockSpec((t, t), lambda i: (0, 0)),
            pl.BlockSpec((1, 1), lambda i: (0, 0)),
        ],
        out_specs=pl.BlockSpec((_BB, t, d), lambda i: (i, 0, 0)),
        out_shape=jax.ShapeDtypeStruct((b, t, d), jnp.float32),
    )(x, Wq, Wk, Wv, jnp.asarray(_D8M), g)


# speedup vs baseline: 1.0186x; 1.0186x over previous
import jax
import jax.numpy as jnp
import numpy as np
from jax.experimental import pallas as pl

_T = 128
_D = 64
_TOPK = 8
_MAXR = 255.0

_BB = 64


def _make_d8m():
    i = np.arange(_T)
    d = np.abs(i[:, None] - i[None, :]).astype(np.float32)
    decay = np.float32(1.0) - (np.float32(0.1) * d) / np.float32(128.0)
    tril = i[None, :] <= i[:, None]
    return np.where(tril, decay * np.float32(0.125),
                    np.float32(0.0)).astype(np.float32)


_D8M = _make_d8m()


def _head_body(x_ref, wq_ref, wk_ref, wv_ref, d8m_ref, g_ref, out_ref):
    x = x_ref[...].reshape(_BB * _T, _D)
    q = jnp.dot(x, wq_ref[...], preferred_element_type=jnp.float32)
    k = jnp.dot(x, wk_ref[...], preferred_element_type=jnp.float32)
    v = jnp.dot(x, wv_ref[...], preferred_element_type=jnp.float32)
    q = q.reshape(_BB, _T, _D)
    k = k.reshape(_BB, _T, _D)
    v = v.reshape(_BB, _T, _D) * (1.0 / g_ref[0, 0])

    s = jax.lax.dot_general(
        q, k, (((2,), (2,)), ((0,), (0,))),
        preferred_element_type=jnp.float32)

    f = jnp.maximum(s, 0.0) * d8m_ref[...][None]

    thresh = jnp.max(f, axis=-1, keepdims=True)
    m = thresh
    for _ in range(_TOPK - 1):
        thresh = jnp.max(jnp.where(f < thresh, f, 0.0),
                         axis=-1, keepdims=True)

    r = _MAXR / (m + 1e-6)
    norm = jnp.floor(f * r)
    w = jnp.where(f >= thresh, norm - jnp.where(norm > 127.5, 256.0, 0.0),
                  0.0)

    out_ref[...] = jax.lax.dot_general(
        w, v, (((2,), (1,)), ((0,), (0,))),
        preferred_element_type=jnp.float32)


def kernel(x, Wk, Wq, Wv, gamma):
    b, t, d = x.shape
    g = jnp.reshape(gamma, (1, 1)).astype(jnp.float32)
    return pl.pallas_call(
        _head_body,
        grid=(b // _BB,),
        in_specs=[
            pl.BlockSpec((_BB, t, d), lambda i: (i, 0, 0)),
            pl.BlockSpec((d, d), lambda i: (0, 0)),
            pl.BlockSpec((d, d), lambda i: (0, 0)),
            pl.BlockSpec((d, d), lambda i: (0, 0)),
            pl.BlockSpec((t, t), lambda i: (0, 0)),
            pl.BlockSpec((1, 1), lambda i: (0, 0)),
        ],
        out_specs=pl.BlockSpec((_BB, t, d), lambda i: (i, 0, 0)),
        out_shape=jax.ShapeDtypeStruct((b, t, d), jnp.float32),
    )(x, Wq, Wk, Wv, jnp.asarray(_D8M), g)


# speedup vs baseline: 1.2557x; 1.2327x over previous
import jax
import jax.numpy as jnp
import numpy as np
from jax.experimental import pallas as pl

_T = 128
_D = 64
_TOPK = 8
_MAXR = 255.0

_BB = 64


def _make_d8m():
    i = np.arange(_T)
    d = np.abs(i[:, None] - i[None, :]).astype(np.float32)
    decay = np.float32(1.0) - (np.float32(0.1) * d) / np.float32(128.0)
    tril = i[None, :] <= i[:, None]
    return np.where(tril, decay * np.float32(0.125),
                    np.float32(0.0)).astype(np.float32)


_D8M = _make_d8m()


def _head_body(x_ref, wq_ref, wk_ref, wv_ref, d8m_ref, g_ref, out_ref):
    x = x_ref[...].reshape(_BB * _T, _D)
    q = jnp.dot(x, wq_ref[...], preferred_element_type=jnp.float32)
    k = jnp.dot(x, wk_ref[...], preferred_element_type=jnp.float32)
    v = jnp.dot(x, wv_ref[...], preferred_element_type=jnp.float32)
    q = q.reshape(_BB, _T, _D)
    k = k.reshape(_BB, _T, _D)
    v = v.reshape(_BB, _T, _D) * (1.0 / g_ref[0, 0])

    st = jax.lax.dot_general(
        k, q, (((2,), (2,)), ((0,), (0,))),
        preferred_element_type=jnp.float32)

    f = jnp.maximum(st, 0.0) * d8m_ref[...][None]

    thresh = jnp.max(f, axis=1, keepdims=True)
    m = thresh
    for _ in range(_TOPK - 1):
        thresh = jnp.max(jnp.where(f < thresh, f, 0.0),
                         axis=1, keepdims=True)

    r = _MAXR / (m + 1e-6)
    norm = jnp.floor(f * r)
    w = jnp.where(f >= thresh, norm - jnp.where(norm > 127.5, 256.0, 0.0),
                  0.0)

    out_ref[...] = jax.lax.dot_general(
        w, v, (((1,), (1,)), ((0,), (0,))),
        preferred_element_type=jnp.float32)


def kernel(x, Wk, Wq, Wv, gamma):
    b, t, d = x.shape
    g = jnp.reshape(gamma, (1, 1)).astype(jnp.float32)
    return pl.pallas_call(
        _head_body,
        grid=(b // _BB,),
        in_specs=[
            pl.BlockSpec((_BB, t, d), lambda i: (i, 0, 0)),
            pl.BlockSpec((d, d), lambda i: (0, 0)),
            pl.BlockSpec((d, d), lambda i: (0, 0)),
            pl.BlockSpec((d, d), lambda i: (0, 0)),
            pl.BlockSpec((t, t), lambda i: (0, 0)),
            pl.BlockSpec((1, 1), lambda i: (0, 0)),
        ],
        out_specs=pl.BlockSpec((_BB, t, d), lambda i: (i, 0, 0)),
        out_shape=jax.ShapeDtypeStruct((b, t, d), jnp.float32),
    )(x, Wq, Wk, Wv, jnp.asarray(_D8M.T.copy()), g)
